# Initial kernel scaffold; baseline (speedup 1.0000x reference)
import jax
import jax.numpy as jnp
from jax.experimental import pallas as pl


def kernel(x, edge_index, params):
    raise NotImplementedError("write your pallas kernel here")


# speedup vs baseline: 2.6223x; 2.6223x over previous
import functools

import jax
import jax.numpy as jnp
from jax import lax
from jax.experimental import pallas as pl
from jax.experimental.pallas import tpu as pltpu
from jax.experimental.pallas import tpu_sc as plsc

N = 10000
E = 320000
NW = 32
EPT = E // NW
C = 80
NCH = EPT // C
RPT = N // 16

_mesh = plsc.VectorSubcoreMesh(core_axis_name="c", subcore_axis_name="s")


@functools.partial(
    pl.kernel,
    out_type=jax.ShapeDtypeStruct((E, 64), jnp.float32),
    mesh=_mesh,
    scratch_types=[
        pltpu.VMEM((C,), jnp.int32),
        pltpu.VMEM((C,), jnp.int32),
        pltpu.VMEM((C, 64), jnp.float32),
        pltpu.VMEM((C, 64), jnp.float32),
        pltpu.SemaphoreType.DMA,
        pltpu.SemaphoreType.DMA,
    ],
    compiler_params=pltpu.CompilerParams(use_tc_tiling_on_sc=False),
)
def _gather_add(dst_h, src_h, pd_h, ps_h, g_h, idxd, idxs, bufa, bufb, sem0, sem1):
    c = lax.axis_index("c")
    s = lax.axis_index("s")
    tbase = (c * 16 + s) * EPT

    def chunk(k, carry):
        base = tbase + k * C
        pltpu.sync_copy(dst_h.at[pl.ds(base, C)], idxd)
        pltpu.sync_copy(src_h.at[pl.ds(base, C)], idxs)
        cpa = pltpu.async_copy(pd_h.at[idxd], bufa, sem0)
        cpb = pltpu.async_copy(ps_h.at[idxs], bufb, sem1)
        cpa.wait()
        cpb.wait()

        def row(i, carry2):
            for j in range(4):
                sl = pl.ds(j * 16, 16)
                bufa[i, sl] = bufa[i, sl] + bufb[i, sl]
            return carry2

        lax.fori_loop(0, C, row, 0)
        pltpu.sync_copy(bufa, g_h.at[pl.ds(base, C)])
        return carry

    lax.fori_loop(0, NCH, chunk, 0)


@functools.partial(
    pl.kernel,
    out_type=jax.ShapeDtypeStruct((2, N, 64), jnp.float32),
    mesh=_mesh,
    scratch_types=[
        pltpu.VMEM((C,), jnp.int32),
        pltpu.VMEM((C, 64), jnp.float32),
        pltpu.VMEM_SHARED((N, 64), jnp.float32),
        pltpu.SemaphoreType.DMA,
    ],
    compiler_params=pltpu.CompilerParams(use_tc_tiling_on_sc=False),
)
def _segsum(dst_h, val_h, zeros_h, out_h, idx, buf, accum, sem):
    c = lax.axis_index("c")
    s = lax.axis_index("s")
    rsl = pl.ds(s * RPT, RPT)
    pltpu.sync_copy(zeros_h.at[rsl], accum.at[rsl])
    plsc.subcore_barrier()
    tbase = (c * 16 + s) * EPT

    def chunk(k, carry):
        base = tbase + k * C
        pltpu.sync_copy(dst_h.at[pl.ds(base, C)], idx)
        pltpu.sync_copy(val_h.at[pl.ds(base, C)], buf)
        pltpu.sync_copy(buf, accum.at[idx], add=True)
        return carry

    lax.fori_loop(0, NCH, chunk, 0)
    plsc.subcore_barrier()
    pltpu.sync_copy(accum.at[rsl], out_h.at[c, rsl])


@functools.partial(
    pl.kernel,
    out_type=jax.ShapeDtypeStruct((2, N, 16), jnp.float32),
    mesh=_mesh,
    scratch_types=[
        pltpu.VMEM((C,), jnp.int32),
        pltpu.VMEM((C, 16), jnp.float32),
        pltpu.VMEM_SHARED((N, 16), jnp.float32),
        pltpu.SemaphoreType.DMA,
    ],
    compiler_params=pltpu.CompilerParams(use_tc_tiling_on_sc=False),
)
def _degree(dst_h, ones_h, zeros_h, out_h, idx, buf, accum, sem):
    c = lax.axis_index("c")
    s = lax.axis_index("s")
    rsl = pl.ds(s * RPT, RPT)
    pltpu.sync_copy(zeros_h.at[rsl], accum.at[rsl])
    pltpu.sync_copy(ones_h, buf)
    plsc.subcore_barrier()
    tbase = (c * 16 + s) * EPT

    def chunk(k, carry):
        base = tbase + k * C
        pltpu.sync_copy(dst_h.at[pl.ds(base, C)], idx)
        pltpu.sync_copy(buf, accum.at[idx], add=True)
        return carry

    lax.fori_loop(0, NCH, chunk, 0)
    plsc.subcore_barrier()
    pltpu.sync_copy(accum.at[rsl], out_h.at[c, rsl])


TILE_N = 2000
TILE_E = 4000


def _tables_body(x_ref, wd_ref, ws_ref, bd_ref, pd_ref, ps_ref):
    xb = x_ref[...]
    pd_ref[...] = jnp.dot(xb, wd_ref[...], preferred_element_type=jnp.float32) + bd_ref[...]
    ps_ref[...] = jnp.dot(xb, ws_ref[...], preferred_element_type=jnp.float32)


def _mlp_body(g_ref, w1_ref, b1_ref, w2_ref, b2_ref, o_ref):
    h1 = jnp.maximum(g_ref[...], 0.0)
    h2 = jnp.maximum(
        jnp.dot(h1, w1_ref[...], preferred_element_type=jnp.float32) + b1_ref[...], 0.0)
    o_ref[...] = jnp.maximum(
        jnp.dot(h2, w2_ref[...], preferred_element_type=jnp.float32) + b2_ref[...], 0.0)


def _mid_body(sp_ref, degp_ref, w3_ref, b3_ref, wd2_ref, ws2_ref, bd2_ref,
              pd2_ref, ps2_ref):
    ssum = sp_ref[0] + sp_ref[1]
    deg = degp_ref[0, :, 0:1] + degp_ref[1, :, 0:1]
    h = jnp.maximum(
        jnp.dot(ssum, w3_ref[...], preferred_element_type=jnp.float32) + deg * b3_ref[...],
        0.0)
    pd2_ref[...] = jnp.dot(h, wd2_ref[...], preferred_element_type=jnp.float32) + bd2_ref[...]
    ps2_ref[...] = jnp.dot(h, ws2_ref[...], preferred_element_type=jnp.float32)


def _final_body(sp_ref, degp_ref, w3_ref, b3_ref, o_ref):
    ssum = sp_ref[0] + sp_ref[1]
    deg = degp_ref[0, :, 0:1] + degp_ref[1, :, 0:1]
    o_ref[...] = jnp.dot(ssum, w3_ref[...], preferred_element_type=jnp.float32) + deg * b3_ref[...]


def _tables(x, wd, ws, bd):
    din = x.shape[1]
    return pl.pallas_call(
        _tables_body,
        grid=(N // TILE_N,),
        in_specs=[
            pl.BlockSpec((TILE_N, din), lambda i: (i, 0)),
            pl.BlockSpec((din, 64), lambda i: (0, 0)),
            pl.BlockSpec((din, 64), lambda i: (0, 0)),
            pl.BlockSpec((1, 64), lambda i: (0, 0)),
        ],
        out_specs=[
            pl.BlockSpec((TILE_N, 64), lambda i: (i, 0)),
            pl.BlockSpec((TILE_N, 64), lambda i: (i, 0)),
        ],
        out_shape=[
            jax.ShapeDtypeStruct((N, 64), jnp.float32),
            jax.ShapeDtypeStruct((N, 64), jnp.float32),
        ],
    )(x, wd, ws, bd)


def _mlp(g, w1, b1, w2, b2):
    return pl.pallas_call(
        _mlp_body,
        grid=(E // TILE_E,),
        in_specs=[
            pl.BlockSpec((TILE_E, 64), lambda i: (i, 0)),
            pl.BlockSpec((64, 64), lambda i: (0, 0)),
            pl.BlockSpec((1, 64), lambda i: (0, 0)),
            pl.BlockSpec((64, 64), lambda i: (0, 0)),
            pl.BlockSpec((1, 64), lambda i: (0, 0)),
        ],
        out_specs=pl.BlockSpec((TILE_E, 64), lambda i: (i, 0)),
        out_shape=jax.ShapeDtypeStruct((E, 64), jnp.float32),
    )(g, w1, b1, w2, b2)


def _mid(sp, degp, w3, b3, wd2, ws2, bd2):
    return pl.pallas_call(
        _mid_body,
        grid=(N // TILE_N,),
        in_specs=[
            pl.BlockSpec((2, TILE_N, 64), lambda i: (0, i, 0)),
            pl.BlockSpec((2, TILE_N, 16), lambda i: (0, i, 0)),
            pl.BlockSpec((64, 64), lambda i: (0, 0)),
            pl.BlockSpec((1, 64), lambda i: (0, 0)),
            pl.BlockSpec((64, 64), lambda i: (0, 0)),
            pl.BlockSpec((64, 64), lambda i: (0, 0)),
            pl.BlockSpec((1, 64), lambda i: (0, 0)),
        ],
        out_specs=[
            pl.BlockSpec((TILE_N, 64), lambda i: (i, 0)),
            pl.BlockSpec((TILE_N, 64), lambda i: (i, 0)),
        ],
        out_shape=[
            jax.ShapeDtypeStruct((N, 64), jnp.float32),
            jax.ShapeDtypeStruct((N, 64), jnp.float32),
        ],
    )(sp, degp, w3, b3, wd2, ws2, bd2)


def _final(sp, degp, w3, b3):
    return pl.pallas_call(
        _final_body,
        grid=(N // TILE_N,),
        in_specs=[
            pl.BlockSpec((2, TILE_N, 64), lambda i: (0, i, 0)),
            pl.BlockSpec((2, TILE_N, 16), lambda i: (0, i, 0)),
            pl.BlockSpec((64, 128), lambda i: (0, 0)),
            pl.BlockSpec((1, 128), lambda i: (0, 0)),
        ],
        out_specs=pl.BlockSpec((TILE_N, 128), lambda i: (i, 0)),
        out_shape=jax.ShapeDtypeStruct((N, 128), jnp.float32),
    )(sp, degp, w3, b3)


def kernel(x, edge_index, params):
    src = edge_index[0]
    dst = edge_index[1]
    (W0, b0), (W1, b1), (W2, b2), (W3, b3) = params["conv1"]
    (V0, c0), (V1, c1), (V2, c2), (V3, c3) = params["conv2"]
    W0i, W0j = W0[:128], W0[128:]
    V0i, V0j = V0[:64], V0[64:]

    zeros64 = jnp.zeros((N, 64), jnp.float32)
    zeros16 = jnp.zeros((N, 16), jnp.float32)
    onesC = jnp.ones((C, 16), jnp.float32)

    degp = _degree(dst, onesC, zeros16)

    pd1, ps1 = _tables(x, W0i - W0j, W0j, b0.reshape(1, 64))
    g1 = _gather_add(dst, src, pd1, ps1)
    h3_1 = _mlp(g1, W1, b1.reshape(1, 64), W2, b2.reshape(1, 64))
    s1p = _segsum(dst, h3_1, zeros64)

    pd2, ps2 = _mid(s1p, degp, W3, b3.reshape(1, 64),
                    V0i - V0j, V0j, c0.reshape(1, 64))
    g2 = _gather_add(dst, src, pd2, ps2)
    h3_2 = _mlp(g2, V1, c1.reshape(1, 64), V2, c2.reshape(1, 64))
    s2p = _segsum(dst, h3_2, zeros64)

    return _final(s2p, degp, V3, c3.reshape(1, 128))
